# speedup vs baseline: 6.0963x; 6.0963x over previous
import functools

import jax
import jax.numpy as jnp
from jax import lax
from jax.experimental import pallas as pl
from jax.experimental.pallas import tpu as pltpu
from jax.experimental.pallas import tpu_sc as plsc

NC = 2
NS = 16
EBLK = 128


def _sc_aggregate(x, src, dst):
    n, d = x.shape
    e = src.shape[0]
    nblk = e // EBLK
    nw = NC * NS
    jmax = nblk // nw
    rem = nblk - jmax * nw

    rows_per = (n // NS) & ~7
    tail_r0 = rows_per * NS
    tail_n = n - tail_r0

    mesh = plsc.VectorSubcoreMesh(core_axis_name="c", subcore_axis_name="s")

    @functools.partial(
        pl.kernel,
        mesh=mesh,
        out_type=jax.ShapeDtypeStruct((NC, n, d), jnp.float32),
        scratch_types=[
            pltpu.VMEM((EBLK,), jnp.int32),
            pltpu.VMEM((EBLK,), jnp.int32),
            pltpu.VMEM((EBLK, d), jnp.float32),
            pltpu.VMEM_SHARED((n, d), jnp.float32),
        ],
    )
    def sc_kernel(x_hbm, src_hbm, dst_hbm, out_hbm, src_v, dst_v, msgs_v, agg_sh):
        c = lax.axis_index("c")
        s = lax.axis_index("s")
        w = c * NS + s

        pltpu.sync_copy(x_hbm.at[pl.ds(s * rows_per, rows_per)],
                        agg_sh.at[pl.ds(s * rows_per, rows_per)])
        if tail_n:
            @pl.when(s == 0)
            def _():
                pltpu.sync_copy(x_hbm.at[pl.ds(tail_r0, tail_n)],
                                agg_sh.at[pl.ds(tail_r0, tail_n)])
        plsc.subcore_barrier()

        def do_block(b):
            off = b * EBLK
            pltpu.sync_copy(src_hbm.at[pl.ds(off, EBLK)], src_v)
            pltpu.sync_copy(dst_hbm.at[pl.ds(off, EBLK)], dst_v)
            pltpu.sync_copy(x_hbm.at[src_v], msgs_v)
            pltpu.sync_copy(msgs_v, agg_sh.at[dst_v], add=True)

        @pl.loop(0, jmax)
        def _(j):
            do_block(j * nw + w)

        if rem:
            @pl.when(w < rem)
            def _():
                do_block(jmax * nw + w)

        plsc.subcore_barrier()

        pltpu.sync_copy(agg_sh.at[pl.ds(s * rows_per, rows_per)],
                        out_hbm.at[c, pl.ds(s * rows_per, rows_per)])
        if tail_n:
            @pl.when(s == 0)
            def _():
                pltpu.sync_copy(agg_sh.at[pl.ds(tail_r0, tail_n)],
                                out_hbm.at[c, pl.ds(tail_r0, tail_n)])

    return sc_kernel(x, src, dst)


def _tc_body(scale_ref, x_ref, p0_ref, p1_ref, w1_ref, b1_ref, w2_ref, b2_ref,
             o_ref):
    h = x_ref[...] * scale_ref[0, 0] + p0_ref[...] + p1_ref[...]
    h = jnp.dot(h, w1_ref[...], preferred_element_type=jnp.float32,
                precision=lax.Precision.HIGHEST) + b1_ref[...]
    h = jnp.maximum(h, 0.0)
    o_ref[...] = jnp.dot(h, w2_ref[...], preferred_element_type=jnp.float32,
                         precision=lax.Precision.HIGHEST) + b2_ref[...]


def _tc_mlp(x, p0, p1, W1, b1, W2, b2, eps):
    n, d = x.shape
    blk = 1000
    grid = (n // blk,)
    scale = (eps - 1.0).reshape(1, 1)
    return pl.pallas_call(
        _tc_body,
        grid=grid,
        in_specs=[
            pl.BlockSpec((1, 1), lambda i: (0, 0)),
            pl.BlockSpec((blk, d), lambda i: (i, 0)),
            pl.BlockSpec((blk, d), lambda i: (i, 0)),
            pl.BlockSpec((blk, d), lambda i: (i, 0)),
            pl.BlockSpec((d, d), lambda i: (0, 0)),
            pl.BlockSpec((1, d), lambda i: (0, 0)),
            pl.BlockSpec((d, d), lambda i: (0, 0)),
            pl.BlockSpec((1, d), lambda i: (0, 0)),
        ],
        out_specs=pl.BlockSpec((blk, d), lambda i: (i, 0)),
        out_shape=jax.ShapeDtypeStruct((n, d), jnp.float32),
    )(scale, x, p0, p1, W1, b1.reshape(1, d), W2, b2.reshape(1, d))


def kernel(x, edge_index, W1, b1, W2, b2, eps):
    src = edge_index[0]
    dst = edge_index[1]
    partials = _sc_aggregate(x, src, dst)
    return _tc_mlp(x, partials[0], partials[1], W1, b1, W2, b2, eps)
